# baseline (speedup 1.0000x reference)
import jax
import jax.numpy as jnp
from jax.experimental import pallas as pl
from jax.experimental.pallas import tpu as pltpu

_N, _D, _F, _K = 2048, 768, 16384, 64
_R = 128
_T = _N // _R
_C = 512
_NC = _F // _C
_C2 = 1024
_NC2 = _F // _C2
_CD = 512
_NCD = _F // _CD


def _ae_body(b_ref, x_ref, w_hbm, o_ref,
             w_ref, hi_ref, lo_ref, xp_ref, thr_ref, sem):
    @pl.when(pl.program_id(0) == 0)
    def _load_w():
        cp = pltpu.make_async_copy(w_hbm, w_ref, sem)
        cp.start()
        cp.wait()

    _SENT = jnp.int16(-32768)
    xc = x_ref[...] - b_ref[...]
    xp = xp_ref[...]
    thrf = thr_ref[...]

    def _enc(c, carry):
        w_c = w_ref[:, pl.ds(c * _C, _C)]
        f = jax.lax.dot_general(xc, w_c, (((1,), (0,)), ((), ())),
                                preferred_element_type=jnp.float32)
        ab = (jax.lax.bitcast_convert_type(f, jnp.int32)
              & jnp.int32(0x7FFFFFFF))
        sl = pl.ds(c * _C, _C)
        hi_ref[:, sl] = jax.lax.shift_right_logical(ab, 16).astype(jnp.int16)
        lo_ref[:, sl] = ((ab & jnp.int32(0xFFFF)) - 32768).astype(jnp.int16)
        return carry
    jax.lax.fori_loop(0, _NC, _enc, 0)

    one = jnp.int16(1)
    zero = jnp.int16(0)

    def _fold_max(v):
        v = v.astype(jnp.int32)
        v = jnp.maximum(v[:, :512], v[:, 512:])
        v = jnp.maximum(v[:, :256], v[:, 256:])
        return jnp.maximum(v[:, :128], v[:, 128:])

    def _fold_sum(p):
        p = p[:, :512] + p[:, 512:]
        p = p[:, :256] + p[:, 256:]
        return p[:, :128] + p[:, 128:]

    def _count16(mid16):
        acc = jnp.zeros((_R, 128), jnp.int16)
        for c in range(_NC2):
            v = hi_ref[:, c * _C2:(c + 1) * _C2]
            acc = acc + _fold_sum(jnp.where(v >= mid16, one, zero))
        return jnp.sum(acc.astype(jnp.int32), axis=1, keepdims=True)

    def _dec_chunk(c, acc):
        w_c = w_ref[:, pl.ds(c * _CD, _CD)]
        f = jax.lax.dot_general(xp, w_c, (((1,), (0,)), ((), ())),
                                preferred_element_type=jnp.float32)
        m = jnp.where(jnp.abs(f) >= thrf, f, 0.0)
        return acc + jax.lax.dot_general(
            m, w_c, (((1,), (1,)), ((), ())),
            preferred_element_type=jnp.float32)

    def _stepA(j, carry):
        lo, hi, chi, acc = carry
        mid = lo + jax.lax.shift_right_logical(hi - lo, 1)
        cnt = _count16(mid.astype(jnp.int16))
        take = cnt >= _K
        lo = jnp.where(take, mid, lo)
        hi = jnp.where(take, hi, mid)
        chi = jnp.where(take, chi, cnt)
        acc = _dec_chunk(j, acc)
        return lo, hi, chi, acc
    t16, _, c_hi, acc = jax.lax.fori_loop(
        0, 15, _stepA,
        (jnp.zeros((_R, 1), jnp.int32), jnp.full((_R, 1), 0x7F80, jnp.int32),
         jnp.zeros((_R, 1), jnp.int32),
         jnp.zeros((_R, _D), jnp.float32)))
    t16_16 = t16.astype(jnp.int16)
    r_need = _K - c_hi

    def _key(c, carry):
        sl = pl.ds(c * _C2, _C2)
        bnd = hi_ref[:, sl] == t16_16
        hi_ref[:, sl] = jnp.where(bnd, lo_ref[:, sl], _SENT)
        return _dec_chunk(15 + c, carry)
    acc = jax.lax.fori_loop(0, _NC2, _key, acc)

    mx = jnp.full((_R, 128), -32768, jnp.int32)
    for c in range(_NC2):
        v = hi_ref[:, c * _C2:(c + 1) * _C2]
        mx = jnp.maximum(mx, _fold_max(v))
    m0 = jnp.max(mx, axis=1, keepdims=True)

    def _cond(carry):
        cum, r_need, m_prev, thr_lo = carry
        return jnp.max(r_need - cum) > 0

    def _body(carry):
        cum, r_need, m_prev, thr_lo = carry
        act = cum < r_need
        m_prev16 = jnp.where(act, m_prev, jnp.int32(-32769)).astype(jnp.int16)
        mx = jnp.full((_R, 128), -32768, jnp.int32)
        eqc = jnp.zeros((_R, 128), jnp.int16)
        for c in range(_NC2):
            sl = pl.ds(c * _C2, _C2)
            v = hi_ref[:, sl]
            eq = v == m_prev16
            eqc = eqc + _fold_sum(jnp.where(eq, one, zero))
            v = jnp.where(eq, _SENT, v)
            hi_ref[:, sl] = v
            mx = jnp.maximum(mx, _fold_max(v))
        m_new = jnp.max(mx, axis=1, keepdims=True)
        ec = jnp.sum(eqc.astype(jnp.int32), axis=1, keepdims=True)
        cum_new = jnp.where(act, cum + ec, cum)
        done_now = act & (cum_new >= r_need)
        thr_lo = jnp.where(done_now, m_prev, thr_lo)
        m_prev = jnp.where(act, m_new, m_prev)
        return cum_new, r_need, m_prev, thr_lo

    _, _, _, thr_lo = jax.lax.while_loop(
        _cond, _body,
        (jnp.zeros((_R, 1), jnp.int32), r_need, m0,
         jnp.zeros((_R, 1), jnp.int32)))

    tlo = jnp.where(thr_lo == -32768, jnp.int32(0), thr_lo + 32768)

    def _dtail(c, acc):
        return _dec_chunk(15 + _NC2 + c, acc)
    acc = jax.lax.fori_loop(0, _NCD - 15 - _NC2, _dtail, acc)
    o_ref[...] = acc + b_ref[...]

    thr_ref[...] = jax.lax.bitcast_convert_type(
        jax.lax.shift_left(t16, 16) | tlo, jnp.float32)
    xp_ref[...] = xc


def kernel(x, W, b_dec):
    b2 = b_dec.reshape(1, _D)
    last = _T - 1
    return pl.pallas_call(
        _ae_body,
        grid=(_T + 1,),
        in_specs=[
            pl.BlockSpec((1, _D), lambda i: (0, 0)),
            pl.BlockSpec((_R, _D), lambda i: (jnp.minimum(i, last), 0)),
            pl.BlockSpec(memory_space=pl.ANY),
        ],
        out_specs=pl.BlockSpec((_R, _D), lambda i: (jnp.maximum(i - 1, 0), 0)),
        out_shape=jax.ShapeDtypeStruct((_N, _D), jnp.float32),
        scratch_shapes=[
            pltpu.VMEM((_D, _F), jnp.float32),
            pltpu.VMEM((_R, _F), jnp.int16),
            pltpu.VMEM((_R, _F), jnp.int16),
            pltpu.VMEM((_R, _D), jnp.float32),
            pltpu.VMEM((_R, 1), jnp.float32),
            pltpu.SemaphoreType.DMA,
        ],
        compiler_params=pltpu.CompilerParams(vmem_limit_bytes=67_043_000),
    )(b2, x, W)


# speedup vs baseline: 1.4073x; 1.4073x over previous
import jax
import jax.numpy as jnp
from jax.experimental import pallas as pl
from jax.experimental.pallas import tpu as pltpu

_N, _D, _F, _K = 2048, 768, 16384, 64
_R = 128
_T = _N // _R
_C = 1024
_NC = _F // _C
_C2 = 2048
_NC2 = _F // _C2
_CD = 512
_NCD = _F // _CD


def _ae_body(b_ref, x_ref, w_hbm, o_ref,
             w_ref, hi_ref, lo_ref, xp_ref, thr_ref, sem):
    @pl.when(pl.program_id(0) == 0)
    def _load_w():
        cp = pltpu.make_async_copy(w_hbm, w_ref, sem)
        cp.start()
        cp.wait()

    xc = x_ref[...] - b_ref[...]
    xp = xp_ref[...]
    thrf = thr_ref[...]

    def _enc(c, carry):
        w_c = w_ref[:, pl.ds(c * _C, _C)]
        f = jax.lax.dot_general(xc, w_c, (((1,), (0,)), ((), ())),
                                preferred_element_type=jnp.float32)
        ab = (jax.lax.bitcast_convert_type(f, jnp.int32)
              & jnp.int32(0x7FFFFFFF))
        sl = pl.ds(c * _C, _C)
        hi_ref[:, sl] = jax.lax.shift_right_logical(ab, 16).astype(jnp.int16)
        lo_ref[:, sl] = ((ab & jnp.int32(0xFFFF)) - 32768).astype(jnp.int16)
        return carry
    jax.lax.fori_loop(0, _NC, _enc, 0)

    def _count16(mid16):
        acc = jnp.zeros((_R, 1), jnp.int32)
        one = jnp.int16(1)
        zero = jnp.int16(0)
        for c in range(_NC2):
            v = hi_ref[:, c * _C2:(c + 1) * _C2]
            p = jnp.where(v >= mid16, one, zero)
            p = p[:, :1024] + p[:, 1024:]
            p = p[:, :512] + p[:, 512:]
            p = p[:, :256] + p[:, 256:]
            p = p[:, :128] + p[:, 128:]
            acc = acc + jnp.sum(p.astype(jnp.int32), axis=1, keepdims=True)
        return acc

    def _dec_chunk(c, acc):
        w_c = w_ref[:, pl.ds(c * _CD, _CD)]
        f = jax.lax.dot_general(xp, w_c, (((1,), (0,)), ((), ())),
                                preferred_element_type=jnp.float32)
        m = jnp.where(jnp.abs(f) >= thrf, f, 0.0)
        return acc + jax.lax.dot_general(
            m, w_c, (((1,), (1,)), ((), ())),
            preferred_element_type=jnp.float32)

    def _stepA(j, carry):
        lo, hi, acc = carry
        mid = lo + jax.lax.shift_right_logical(hi - lo, 1)
        cnt = _count16(mid.astype(jnp.int16))
        take = cnt >= _K
        acc = _dec_chunk(j, acc)
        return jnp.where(take, mid, lo), jnp.where(take, hi, mid), acc
    t16, _, acc = jax.lax.fori_loop(
        0, 15, _stepA,
        (jnp.zeros((_R, 1), jnp.int32), jnp.full((_R, 1), 0x7F80, jnp.int32),
         jnp.zeros((_R, _D), jnp.float32)))
    t16_16 = t16.astype(jnp.int16)

    c_hi = _count16((t16 + 1).astype(jnp.int16))
    r_need = _K - c_hi

    def _key(c, carry):
        sl = pl.ds(c * _C2, _C2)
        bnd = hi_ref[:, sl] == t16_16
        hi_ref[:, sl] = jnp.where(bnd, lo_ref[:, sl], jnp.int16(-32768))
        return carry
    jax.lax.fori_loop(0, _NC2, _key, 0)

    def _stepB(j, carry):
        lo, hi, acc = carry
        mid = lo + jax.lax.shift_right_logical(hi - lo, 1)
        cnt = _count16((mid - 32768).astype(jnp.int16))
        take = cnt >= r_need
        acc = _dec_chunk(15 + j, acc)
        return jnp.where(take, mid, lo), jnp.where(take, hi, mid), acc
    tlo, _, acc = jax.lax.fori_loop(
        0, 16, _stepB,
        (jnp.zeros((_R, 1), jnp.int32), jnp.full((_R, 1), 65536, jnp.int32),
         acc))

    acc = _dec_chunk(_NCD - 1, acc)
    o_ref[...] = acc + b_ref[...]

    thr_ref[...] = jax.lax.bitcast_convert_type(
        jax.lax.shift_left(t16, 16) | tlo, jnp.float32)
    xp_ref[...] = xc


def kernel(x, W, b_dec):
    b2 = b_dec.reshape(1, _D)
    last = _T - 1
    return pl.pallas_call(
        _ae_body,
        grid=(_T + 1,),
        in_specs=[
            pl.BlockSpec((1, _D), lambda i: (0, 0)),
            pl.BlockSpec((_R, _D), lambda i: (jnp.minimum(i, last), 0)),
            pl.BlockSpec(memory_space=pl.ANY),
        ],
        out_specs=pl.BlockSpec((_R, _D), lambda i: (jnp.maximum(i - 1, 0), 0)),
        out_shape=jax.ShapeDtypeStruct((_N, _D), jnp.float32),
        scratch_shapes=[
            pltpu.VMEM((_D, _F), jnp.float32),
            pltpu.VMEM((_R, _F), jnp.int16),
            pltpu.VMEM((_R, _F), jnp.int16),
            pltpu.VMEM((_R, _D), jnp.float32),
            pltpu.VMEM((_R, 1), jnp.float32),
            pltpu.SemaphoreType.DMA,
        ],
        compiler_params=pltpu.CompilerParams(vmem_limit_bytes=66_900_000),
    )(b2, x, W)
